# Initial kernel scaffold; baseline (speedup 1.0000x reference)
import jax
import jax.numpy as jnp
from jax.experimental import pallas as pl


def kernel(x1, x2, x3):
    raise NotImplementedError("write your pallas kernel here")


# speedup vs baseline: 2.6381x; 2.6381x over previous
import functools

import jax
import jax.numpy as jnp
from jax.experimental import pallas as pl

MARGIN_ = 1.0
B_ = 4096
D_ = 16
TILE_ = 512


def _triplet_body(x1_ref, x2_ref, x3_ref, out_ref):
    i = pl.program_id(0)
    a = x1_ref[...]
    p = x2_ref[...]
    n = x3_ref[...]

    ap = jnp.sum((a - p) * (a - p), axis=1)
    a2 = jnp.sum(a * a, axis=1, keepdims=True)
    n2 = jnp.sum(n * n, axis=1)[None, :]
    prod = jax.lax.dot_general(
        a, n, (((1,), (1,)), ((), ())),
        preferred_element_type=jnp.float32)
    d2 = a2 + n2 - 2.0 * prod
    an = jnp.min(d2, axis=1)
    part = jnp.sum(jnp.maximum(ap - an + MARGIN_, 0.0)).reshape(1, 1)

    @pl.when(i == 0)
    def _():
        out_ref[...] = jnp.zeros((1, 1), jnp.float32)

    out_ref[...] += part


@functools.partial(jax.jit, static_argnames=())
def kernel(x1, x2, x3):
    grid = B_ // TILE_
    total = pl.pallas_call(
        _triplet_body,
        grid=(grid,),
        in_specs=[
            pl.BlockSpec((TILE_, D_), lambda i: (i, 0)),
            pl.BlockSpec((TILE_, D_), lambda i: (i, 0)),
            pl.BlockSpec((B_, D_), lambda i: (0, 0)),
        ],
        out_specs=pl.BlockSpec((1, 1), lambda i: (0, 0)),
        out_shape=jax.ShapeDtypeStruct((1, 1), jnp.float32),
    )(x1, x2, x3)
    loss = total[0, 0] / jnp.float32(B_)
    return (loss, jnp.asarray(B_, dtype=jnp.int32))


# speedup vs baseline: 3.1179x; 1.1818x over previous
import functools

import jax
import jax.numpy as jnp
from jax.experimental import pallas as pl
from jax.experimental.pallas import tpu as pltpu

MARGIN_ = 1.0
B_ = 4096
D_ = 16
TILE_ = 512


def _triplet_body(x1_ref, x2_ref, x3_ref, out_ref, naug_ref):
    i = pl.program_id(0)
    nsteps = pl.num_programs(0)
    a = x1_ref[...]
    p = x2_ref[...]

    t = jnp.sum(p * (p - 2.0 * a), axis=1)

    ones = jnp.ones((a.shape[0], 1), jnp.float32)
    a_aug = jnp.concatenate([a, ones], axis=1)

    @pl.when(i == 0)
    def _():
        n = x3_ref[...]
        n2 = jnp.sum(n * n, axis=1, keepdims=True)
        naug_ref[...] = jnp.concatenate([-2.0 * n, n2], axis=1)

    n_aug = naug_ref[...]

    r = jax.lax.dot_general(
        a_aug, n_aug, (((1,), (1,)), ((), ())),
        preferred_element_type=jnp.float32)
    m = jnp.min(r, axis=1)
    part = jnp.sum(jnp.maximum(t - m + MARGIN_, 0.0)).reshape(1, 1)

    @pl.when(i == 0)
    def _():
        out_ref[...] = jnp.zeros((1, 1), jnp.float32)

    out_ref[...] += part

    @pl.when(i == nsteps - 1)
    def _():
        out_ref[...] *= jnp.float32(1.0 / B_)


@functools.partial(jax.jit, static_argnames=())
def kernel(x1, x2, x3):
    grid = B_ // TILE_
    loss = pl.pallas_call(
        _triplet_body,
        grid=(grid,),
        in_specs=[
            pl.BlockSpec((TILE_, D_), lambda i: (i, 0)),
            pl.BlockSpec((TILE_, D_), lambda i: (i, 0)),
            pl.BlockSpec((B_, D_), lambda i: (0, 0)),
        ],
        out_specs=pl.BlockSpec((1, 1), lambda i: (0, 0)),
        out_shape=jax.ShapeDtypeStruct((1, 1), jnp.float32),
        scratch_shapes=[pltpu.VMEM((B_, D_ + 1), jnp.float32)],
    )(x1, x2, x3)
    return (loss.reshape(()), jnp.asarray(B_, dtype=jnp.int32))


# speedup vs baseline: 3.3663x; 1.0797x over previous
import functools

import jax
import jax.numpy as jnp
from jax.experimental import pallas as pl
from jax.experimental.pallas import tpu as pltpu

MARGIN_ = 1.0
B_ = 4096
D_ = 16
TILE_ = 1024


def _triplet_body(x1_ref, x2_ref, x3_ref, out_ref, naug_ref):
    i = pl.program_id(0)
    nsteps = pl.num_programs(0)
    a = x1_ref[...]
    p = x2_ref[...]

    t = jnp.sum(p * (p - 2.0 * a), axis=1)

    ones = jnp.ones((a.shape[0], 1), jnp.float32)
    a_aug = jnp.concatenate([a, ones], axis=1)

    @pl.when(i == 0)
    def _():
        n = x3_ref[...]
        n2 = jnp.sum(n * n, axis=1, keepdims=True)
        naug_ref[...] = jnp.concatenate([-2.0 * n, n2], axis=1)

    n_aug = naug_ref[...]

    r = jax.lax.dot_general(
        a_aug, n_aug, (((1,), (1,)), ((), ())),
        preferred_element_type=jnp.float32)
    m = jnp.min(r, axis=1)
    part = jnp.sum(jnp.maximum(t - m + MARGIN_, 0.0)).reshape(1, 1)

    @pl.when(i == 0)
    def _():
        out_ref[...] = jnp.zeros((1, 1), jnp.float32)

    out_ref[...] += part

    @pl.when(i == nsteps - 1)
    def _():
        out_ref[...] *= jnp.float32(1.0 / B_)


@functools.partial(jax.jit, static_argnames=())
def kernel(x1, x2, x3):
    grid = B_ // TILE_
    loss = pl.pallas_call(
        _triplet_body,
        grid=(grid,),
        in_specs=[
            pl.BlockSpec((TILE_, D_), lambda i: (i, 0)),
            pl.BlockSpec((TILE_, D_), lambda i: (i, 0)),
            pl.BlockSpec((B_, D_), lambda i: (0, 0)),
        ],
        out_specs=pl.BlockSpec((1, 1), lambda i: (0, 0)),
        out_shape=jax.ShapeDtypeStruct((1, 1), jnp.float32),
        scratch_shapes=[pltpu.VMEM((B_, D_ + 1), jnp.float32)],
    )(x1, x2, x3)
    return (loss.reshape(()), jnp.asarray(B_, dtype=jnp.int32))


# speedup vs baseline: 3.4437x; 1.0230x over previous
import functools

import jax
import jax.numpy as jnp
from jax.experimental import pallas as pl
from jax.experimental.pallas import tpu as pltpu

MARGIN_ = 1.0
B_ = 4096
D_ = 16
TILE_ = 2048


def _triplet_body(x1_ref, x2_ref, x3_ref, out_ref, naug_ref):
    i = pl.program_id(0)
    nsteps = pl.num_programs(0)
    a = x1_ref[...]
    p = x2_ref[...]

    t = jnp.sum(p * (p - 2.0 * a), axis=1)

    ones = jnp.ones((a.shape[0], 1), jnp.float32)
    a_aug = jnp.concatenate([a, ones], axis=1)

    @pl.when(i == 0)
    def _():
        n = x3_ref[...]
        n2 = jnp.sum(n * n, axis=1, keepdims=True)
        naug_ref[...] = jnp.concatenate([-2.0 * n, n2], axis=1)

    n_aug = naug_ref[...]

    r = jax.lax.dot_general(
        a_aug, n_aug, (((1,), (1,)), ((), ())),
        preferred_element_type=jnp.float32)
    m = jnp.min(r, axis=1)
    part = jnp.sum(jnp.maximum(t - m + MARGIN_, 0.0)).reshape(1, 1)

    @pl.when(i == 0)
    def _():
        out_ref[...] = jnp.zeros((1, 1), jnp.float32)

    out_ref[...] += part

    @pl.when(i == nsteps - 1)
    def _():
        out_ref[...] *= jnp.float32(1.0 / B_)


@functools.partial(jax.jit, static_argnames=())
def kernel(x1, x2, x3):
    grid = B_ // TILE_
    loss = pl.pallas_call(
        _triplet_body,
        grid=(grid,),
        in_specs=[
            pl.BlockSpec((TILE_, D_), lambda i: (i, 0)),
            pl.BlockSpec((TILE_, D_), lambda i: (i, 0)),
            pl.BlockSpec((B_, D_), lambda i: (0, 0)),
        ],
        out_specs=pl.BlockSpec((1, 1), lambda i: (0, 0)),
        out_shape=jax.ShapeDtypeStruct((1, 1), jnp.float32),
        scratch_shapes=[pltpu.VMEM((B_, D_ + 1), jnp.float32)],
    )(x1, x2, x3)
    return (loss.reshape(()), jnp.asarray(B_, dtype=jnp.int32))


# speedup vs baseline: 3.4930x; 1.0143x over previous
import functools

import jax
import jax.numpy as jnp
from jax.experimental import pallas as pl

MARGIN_ = 1.0
B_ = 4096
D_ = 16
CHUNK_ = 512


def _triplet_body(x1_ref, x2_ref, x3_ref, out_ref):
    a = x1_ref[...]
    p = x2_ref[...]
    n = x3_ref[...]

    t = jnp.sum(p * (p - 2.0 * a), axis=1)

    ones = jnp.ones((B_, 1), jnp.float32)
    a_aug = jnp.concatenate([a, ones], axis=1)
    n2 = jnp.sum(n * n, axis=1, keepdims=True)
    n_aug = jnp.concatenate([-2.0 * n, n2], axis=1)

    mins = []
    for c in range(B_ // CHUNK_):
        r = jax.lax.dot_general(
            a_aug[c * CHUNK_:(c + 1) * CHUNK_], n_aug,
            (((1,), (1,)), ((), ())),
            preferred_element_type=jnp.float32)
        mins.append(jnp.min(r, axis=1))
    m = jnp.concatenate(mins, axis=0)

    total = jnp.sum(jnp.maximum(t - m + MARGIN_, 0.0))
    out_ref[...] = (total * jnp.float32(1.0 / B_)).reshape(1, 1)


@functools.partial(jax.jit, static_argnames=())
def kernel(x1, x2, x3):
    loss = pl.pallas_call(
        _triplet_body,
        out_shape=jax.ShapeDtypeStruct((1, 1), jnp.float32),
    )(x1, x2, x3)
    return (loss.reshape(()), jnp.asarray(B_, dtype=jnp.int32))
